# initial kernel scaffold (unmeasured)
import jax
import jax.numpy as jnp
from jax import lax
from jax.experimental import pallas as pl
from jax.experimental.pallas import tpu as pltpu

N_DEV = 4
SQ = 256
SKV = 4096
HQ = 8
DH = 128
D = HQ * DH
SCALE = 0.08838834764831843


def _body(
    x_ref, wq_ref, wo_ref, k_ref, v_ref,
    out_ref,
    qall,
    qag_send_sems, qag_recv_sems,
    pout_send, pstat_send,
    pout_recv, pstat_recv,
    pout_ssem, pstat_ssem,
    pout_rsem, pstat_rsem,
    myout, mystat,
):
    my = lax.axis_index("i")
    right = (my + 1) % N_DEV

    barrier = pltpu.get_barrier_semaphore()
    for off in (1, 2, 3):
        pl.semaphore_signal(
            barrier, inc=1,
            device_id=((my + off) % N_DEV,),
            device_id_type=pltpu.DeviceIdType.MESH,
        )
    pl.semaphore_wait(barrier, N_DEV - 1)

    q = jnp.dot(x_ref[...], wq_ref[...], preferred_element_type=jnp.float32)
    qall[pl.ds(my * SQ, SQ), :] = q.astype(jnp.bfloat16)

    for h in range(N_DEV - 1):
        src = (my - h) % N_DEV
        rdma = pltpu.make_async_remote_copy(
            src_ref=qall.at[pl.ds(src * SQ, SQ), :],
            dst_ref=qall.at[pl.ds(src * SQ, SQ), :],
            send_sem=qag_send_sems.at[h],
            recv_sem=qag_recv_sems.at[h],
            device_id=(right,),
            device_id_type=pltpu.DeviceIdType.MESH,
        )
        rdma.start()
        rdma.wait()

    def partial(qidx):
        q_blk = qall[pl.ds(qidx * SQ, SQ), :]
        outs, ms, ls = [], [], []
        for h in range(HQ):
            qh = q_blk[:, h * DH:(h + 1) * DH]
            kh = k_ref[:, h * DH:(h + 1) * DH]
            vh = v_ref[:, h * DH:(h + 1) * DH]
            s = lax.dot_general(
                qh, kh, (((1,), (1,)), ((), ())),
                preferred_element_type=jnp.float32,
            ) * SCALE
            m = jnp.max(s, axis=1, keepdims=True)
            p = jnp.exp(s - m)
            l = jnp.sum(p, axis=1, keepdims=True)
            o = lax.dot_general(
                p.astype(jnp.bfloat16), vh, (((1,), (0,)), ((), ())),
                preferred_element_type=jnp.float32,
            )
            outs.append(o)
            ms.append(m)
            ls.append(l)
        return outs, ms, ls

    outs, ms, ls = partial(my)
    for h in range(HQ):
        myout[:, h * DH:(h + 1) * DH] = outs[h]
        mystat[:, h:h + 1] = ms[h]
        mystat[:, HQ + h:HQ + h + 1] = ls[h]

    for s in range(1, N_DEV):
        qidx = (my - s) % N_DEV
        slot = s - 1
        outs, ms, ls = partial(qidx)
        for h in range(HQ):
            pout_send[slot, :, h * DH:(h + 1) * DH] = outs[h].astype(jnp.bfloat16)
            pstat_send[slot, :, h:h + 1] = ms[h]
            pstat_send[slot, :, HQ + h:HQ + h + 1] = ls[h]
        rdma_o = pltpu.make_async_remote_copy(
            src_ref=pout_send.at[slot],
            dst_ref=pout_recv.at[slot],
            send_sem=pout_ssem.at[slot],
            recv_sem=pout_rsem.at[slot],
            device_id=(qidx,),
            device_id_type=pltpu.DeviceIdType.MESH,
        )
        rdma_s = pltpu.make_async_remote_copy(
            src_ref=pstat_send.at[slot],
            dst_ref=pstat_recv.at[slot],
            send_sem=pstat_ssem.at[slot],
            recv_sem=pstat_rsem.at[slot],
            device_id=(qidx,),
            device_id_type=pltpu.DeviceIdType.MESH,
        )
        rdma_o.start()
        rdma_s.start()
        rdma_o.wait()
        rdma_s.wait()

    m_p = [mystat[:, 0:HQ]] + [pstat_recv[sl, :, 0:HQ] for sl in range(3)]
    l_p = [mystat[:, HQ:2 * HQ]] + [pstat_recv[sl, :, HQ:2 * HQ] for sl in range(3)]
    mstar = jnp.maximum(jnp.maximum(m_p[0], m_p[1]), jnp.maximum(m_p[2], m_p[3]))
    alpha = [jnp.exp(mp - mstar) for mp in m_p]
    lstar = sum(a * lp for a, lp in zip(alpha, l_p))

    heads = []
    for h in range(HQ):
        hs = slice(h * DH, (h + 1) * DH)
        acc = myout[:, hs] * alpha[0][:, h:h + 1]
        for sl in range(3):
            acc = acc + (
                pout_recv[sl, :, hs].astype(jnp.float32) * alpha[sl + 1][:, h:h + 1]
            )
        heads.append((acc / lstar[:, h:h + 1]).astype(jnp.bfloat16))
    attn = jnp.concatenate(heads, axis=1)
    out_ref[...] = jnp.dot(attn, wo_ref[...], preferred_element_type=jnp.float32)


def kernel(x, Wq, Wo, K_ext, V_ext):
    xb = x[0].astype(jnp.bfloat16)
    wqb = Wq.astype(jnp.bfloat16)
    wob = Wo.astype(jnp.bfloat16)
    kb = K_ext[0].reshape(SKV, D).astype(jnp.bfloat16)
    vb = V_ext[0].reshape(SKV, D).astype(jnp.bfloat16)

    out = pl.pallas_call(
        _body,
        out_shape=jax.ShapeDtypeStruct((SQ, D), jnp.float32),
        in_specs=[pl.BlockSpec(memory_space=pltpu.VMEM)] * 5,
        out_specs=pl.BlockSpec(memory_space=pltpu.VMEM),
        scratch_shapes=[
            pltpu.VMEM((N_DEV * SQ, D), jnp.bfloat16),
            pltpu.SemaphoreType.DMA((N_DEV - 1,)),
            pltpu.SemaphoreType.DMA((N_DEV - 1,)),
            pltpu.VMEM((3, SQ, D), jnp.bfloat16),
            pltpu.VMEM((3, SQ, 128), jnp.float32),
            pltpu.VMEM((3, SQ, D), jnp.bfloat16),
            pltpu.VMEM((3, SQ, 128), jnp.float32),
            pltpu.SemaphoreType.DMA((3,)),
            pltpu.SemaphoreType.DMA((3,)),
            pltpu.SemaphoreType.DMA((3,)),
            pltpu.SemaphoreType.DMA((3,)),
            pltpu.VMEM((SQ, D), jnp.float32),
            pltpu.VMEM((SQ, 128), jnp.float32),
        ],
        compiler_params=pltpu.CompilerParams(collective_id=0),
    )(xb, wqb, wob, kb, vb)
    return out.reshape(1, SQ, D)


# baseline (device time: 152296 ns/iter reference)
import jax
import jax.numpy as jnp
from jax import lax
from jax.experimental import pallas as pl
from jax.experimental.pallas import tpu as pltpu

N_DEV = 4
SQ = 256
SKV = 4096
HQ = 8
DH = 128
D = HQ * DH
SCALE = 0.08838834764831843


def _body(
    x_ref, wq_ref, wo_ref, k_ref, v_ref,
    out_ref,
    qall,
    qag_send_sems, qag_recv_sems,
    pout_send, pstat_send,
    pout_recv, pstat_recv,
    pout_ssem, pstat_ssem,
    pout_rsem, pstat_rsem,
    myout, mystat,
):
    my = lax.axis_index("i")
    right = (my + 1) % N_DEV

    barrier = pltpu.get_barrier_semaphore()
    for off in (1, 2, 3):
        pl.semaphore_signal(
            barrier, inc=1,
            device_id=((my + off) % N_DEV,),
            device_id_type=pltpu.DeviceIdType.MESH,
        )
    pl.semaphore_wait(barrier, N_DEV - 1)

    q = jnp.dot(x_ref[...], wq_ref[...], preferred_element_type=jnp.float32)
    qall[pl.ds(my * SQ, SQ), :] = q.astype(jnp.bfloat16)

    for h in range(N_DEV - 1):
        src = (my - h) % N_DEV
        rdma = pltpu.make_async_remote_copy(
            src_ref=qall.at[pl.ds(src * SQ, SQ), :],
            dst_ref=qall.at[pl.ds(src * SQ, SQ), :],
            send_sem=qag_send_sems.at[h],
            recv_sem=qag_recv_sems.at[h],
            device_id=(right,),
            device_id_type=pltpu.DeviceIdType.MESH,
        )
        rdma.start()
        rdma.wait()

    def partial(qidx):
        q_blk = qall[pl.ds(qidx * SQ, SQ), :]
        outs, ms, ls = [], [], []
        for h in range(HQ):
            qh = q_blk[:, h * DH:(h + 1) * DH]
            kh = k_ref[:, h * DH:(h + 1) * DH]
            vh = v_ref[:, h * DH:(h + 1) * DH]
            s = lax.dot_general(
                qh, kh, (((1,), (1,)), ((), ())),
                preferred_element_type=jnp.float32,
            ) * SCALE
            m = jnp.max(s, axis=1, keepdims=True)
            p = jnp.exp(s - m)
            l = jnp.sum(p, axis=1, keepdims=True)
            o = lax.dot_general(
                p.astype(jnp.bfloat16), vh, (((1,), (0,)), ((), ())),
                preferred_element_type=jnp.float32,
            )
            outs.append(o)
            ms.append(m)
            ls.append(l)
        return outs, ms, ls

    outs, ms, ls = partial(my)
    for h in range(HQ):
        myout[:, h * DH:(h + 1) * DH] = outs[h]
        mystat[:, h:h + 1] = ms[h]
        mystat[:, HQ + h:HQ + h + 1] = ls[h]

    for s in range(1, N_DEV):
        qidx = (my - s) % N_DEV
        slot = s - 1
        outs, ms, ls = partial(qidx)
        for h in range(HQ):
            pout_send[slot, :, h * DH:(h + 1) * DH] = outs[h].astype(jnp.bfloat16)
            pstat_send[slot, :, h:h + 1] = ms[h]
            pstat_send[slot, :, HQ + h:HQ + h + 1] = ls[h]
        rdma_o = pltpu.make_async_remote_copy(
            src_ref=pout_send.at[slot],
            dst_ref=pout_recv.at[slot],
            send_sem=pout_ssem.at[slot],
            recv_sem=pout_rsem.at[slot],
            device_id=(qidx,),
            device_id_type=pltpu.DeviceIdType.MESH,
        )
        rdma_s = pltpu.make_async_remote_copy(
            src_ref=pstat_send.at[slot],
            dst_ref=pstat_recv.at[slot],
            send_sem=pstat_ssem.at[slot],
            recv_sem=pstat_rsem.at[slot],
            device_id=(qidx,),
            device_id_type=pltpu.DeviceIdType.MESH,
        )
        rdma_o.start()
        rdma_s.start()
        rdma_o.wait()
        rdma_s.wait()

    m_p = [mystat[:, 0:HQ]] + [pstat_recv[sl, :, 0:HQ] for sl in range(3)]
    l_p = [mystat[:, HQ:2 * HQ]] + [pstat_recv[sl, :, HQ:2 * HQ] for sl in range(3)]
    mstar = jnp.maximum(jnp.maximum(m_p[0], m_p[1]), jnp.maximum(m_p[2], m_p[3]))
    alpha = [jnp.exp(mp - mstar) for mp in m_p]
    lstar = sum(a * lp for a, lp in zip(alpha, l_p))

    heads = []
    for h in range(HQ):
        hs = slice(h * DH, (h + 1) * DH)
        acc = myout[:, hs] * alpha[0][:, h:h + 1]
        for sl in range(3):
            acc = acc + (
                pout_recv[sl, :, hs].astype(jnp.float32) * alpha[sl + 1][:, h:h + 1]
            )
        heads.append((acc / lstar[:, h:h + 1]).astype(jnp.bfloat16))
    attn = jnp.concatenate(heads, axis=1)
    out_ref[...] = jnp.dot(attn, wo_ref[...], preferred_element_type=jnp.float32)


def kernel(x, Wq, Wo, K_ext, V_ext):
    xb = x[0].astype(jnp.bfloat16)
    wqb = Wq.astype(jnp.bfloat16)
    wob = Wo.astype(jnp.bfloat16)
    kb = K_ext[0].reshape(SKV, D).astype(jnp.bfloat16)
    vb = V_ext[0].reshape(SKV, D).astype(jnp.bfloat16)

    out = pl.pallas_call(
        _body,
        out_shape=jax.ShapeDtypeStruct((SQ, D), jnp.float32),
        in_specs=[pl.BlockSpec(memory_space=pltpu.VMEM)] * 5,
        out_specs=pl.BlockSpec(memory_space=pltpu.VMEM),
        scratch_shapes=[
            pltpu.VMEM((N_DEV * SQ, D), jnp.bfloat16),
            pltpu.SemaphoreType.DMA((N_DEV - 1,)),
            pltpu.SemaphoreType.DMA((N_DEV - 1,)),
            pltpu.VMEM((3, SQ, D), jnp.bfloat16),
            pltpu.VMEM((3, SQ, 128), jnp.float32),
            pltpu.VMEM((3, SQ, D), jnp.bfloat16),
            pltpu.VMEM((3, SQ, 128), jnp.float32),
            pltpu.SemaphoreType.DMA((3,)),
            pltpu.SemaphoreType.DMA((3,)),
            pltpu.SemaphoreType.DMA((3,)),
            pltpu.SemaphoreType.DMA((3,)),
            pltpu.VMEM((SQ, D), jnp.float32),
            pltpu.VMEM((SQ, 128), jnp.float32),
        ],
        compiler_params=pltpu.CompilerParams(
            collective_id=0,
            vmem_limit_bytes=100 * 1024 * 1024,
        ),
    )(xb, wqb, wob, kb, vb)
    return out.reshape(1, SQ, D)


# device time: 105118 ns/iter; 1.4488x vs baseline; 1.4488x over previous
import jax
import jax.numpy as jnp
from jax import lax
from jax.experimental import pallas as pl
from jax.experimental.pallas import tpu as pltpu

N_DEV = 4
SQ = 256
SKV = 4096
HQ = 8
DH = 128
D = HQ * DH
SCALE = 0.08838834764831843


def _body(
    x_ref, wq_ref, wo_ref, k_ref, v_ref,
    out_ref,
    qall,
    qag_send_sems, qag_recv_sems,
    pout_send, pstat_send,
    pout_recv, pstat_recv,
    pout_ssem, pstat_ssem,
    pout_rsem, pstat_rsem,
    myout, mystat,
):
    my = lax.axis_index("i")
    right = (my + 1) % N_DEV

    barrier = pltpu.get_barrier_semaphore()
    for off in (1, 2, 3):
        pl.semaphore_signal(
            barrier, inc=1,
            device_id=((my + off) % N_DEV,),
            device_id_type=pltpu.DeviceIdType.MESH,
        )
    pl.semaphore_wait(barrier, N_DEV - 1)

    q = jnp.dot(x_ref[...], wq_ref[...], preferred_element_type=jnp.float32)
    qall[pl.ds(my * SQ, SQ), :] = (q * SCALE).astype(jnp.bfloat16)

    def ag_hop(h):
        src = (my - h) % N_DEV
        rdma = pltpu.make_async_remote_copy(
            src_ref=qall.at[pl.ds(src * SQ, SQ), :],
            dst_ref=qall.at[pl.ds(src * SQ, SQ), :],
            send_sem=qag_send_sems.at[h],
            recv_sem=qag_recv_sems.at[h],
            device_id=(right,),
            device_id_type=pltpu.DeviceIdType.MESH,
        )
        rdma.start()
        return rdma

    def partial(qidx):
        q_blk = qall[pl.ds(qidx * SQ, SQ), :]
        outs, ms, ls = [], [], []
        for h in range(HQ):
            qh = q_blk[:, h * DH:(h + 1) * DH]
            kh = k_ref[:, h * DH:(h + 1) * DH]
            vh = v_ref[:, h * DH:(h + 1) * DH]
            s = lax.dot_general(
                qh, kh, (((1,), (1,)), ((), ())),
                preferred_element_type=jnp.float32,
            )
            m = jnp.max(s, axis=1, keepdims=True)
            p = jnp.exp(s - m)
            l = jnp.sum(p, axis=1, keepdims=True)
            o = lax.dot_general(
                p.astype(jnp.bfloat16), vh, (((1,), (0,)), ((), ())),
                preferred_element_type=jnp.float32,
            )
            outs.append(o)
            ms.append(m)
            ls.append(l)
        return outs, jnp.concatenate(ms, axis=1), jnp.concatenate(ls, axis=1)

    ag = [None] * (N_DEV - 1)
    ag[0] = ag_hop(0)

    outs, m_cat, l_cat = partial(my)
    for h in range(HQ):
        myout[:, h * DH:(h + 1) * DH] = outs[h]
    mystat[:, 0:HQ] = m_cat
    mystat[:, HQ:2 * HQ] = l_cat

    psend = []
    for s in range(1, N_DEV):
        qidx = (my - s) % N_DEV
        slot = s - 1
        ag[s - 1].wait_recv()
        if s < N_DEV - 1:
            ag[s] = ag_hop(s)
        outs, m_cat, l_cat = partial(qidx)
        for h in range(HQ):
            pout_send[slot, :, h * DH:(h + 1) * DH] = outs[h].astype(jnp.bfloat16)
        pstat_send[slot, :, 0:HQ] = m_cat
        pstat_send[slot, :, HQ:2 * HQ] = l_cat
        rdma_o = pltpu.make_async_remote_copy(
            src_ref=pout_send.at[slot],
            dst_ref=pout_recv.at[slot],
            send_sem=pout_ssem.at[slot],
            recv_sem=pout_rsem.at[slot],
            device_id=(qidx,),
            device_id_type=pltpu.DeviceIdType.MESH,
        )
        rdma_s = pltpu.make_async_remote_copy(
            src_ref=pstat_send.at[slot],
            dst_ref=pstat_recv.at[slot],
            send_sem=pstat_ssem.at[slot],
            recv_sem=pstat_rsem.at[slot],
            device_id=(qidx,),
            device_id_type=pltpu.DeviceIdType.MESH,
        )
        rdma_o.start()
        rdma_s.start()
        psend.append((rdma_o, rdma_s))

    for h in range(N_DEV - 1):
        ag[h].wait_send()
    for rdma_o, rdma_s in psend:
        rdma_o.wait_send()
        rdma_s.wait_send()
        rdma_o.wait_recv()
        rdma_s.wait_recv()

    m_p = [mystat[:, 0:HQ]] + [pstat_recv[sl, :, 0:HQ] for sl in range(3)]
    l_p = [mystat[:, HQ:2 * HQ]] + [pstat_recv[sl, :, HQ:2 * HQ] for sl in range(3)]
    mstar = jnp.maximum(jnp.maximum(m_p[0], m_p[1]), jnp.maximum(m_p[2], m_p[3]))
    alpha = [jnp.exp(mp - mstar) for mp in m_p]
    lstar = sum(a * lp for a, lp in zip(alpha, l_p))

    heads = []
    for h in range(HQ):
        hs = slice(h * DH, (h + 1) * DH)
        acc = myout[:, hs] * alpha[0][:, h:h + 1]
        for sl in range(3):
            acc = acc + (
                pout_recv[sl, :, hs].astype(jnp.float32) * alpha[sl + 1][:, h:h + 1]
            )
        heads.append((acc / lstar[:, h:h + 1]).astype(jnp.bfloat16))
    attn = jnp.concatenate(heads, axis=1)
    out_ref[...] = jnp.dot(attn, wo_ref[...], preferred_element_type=jnp.float32)


def kernel(x, Wq, Wo, K_ext, V_ext):
    xb = x[0].astype(jnp.bfloat16)
    wqb = Wq.astype(jnp.bfloat16)
    wob = Wo.astype(jnp.bfloat16)
    kb = K_ext[0].reshape(SKV, D).astype(jnp.bfloat16)
    vb = V_ext[0].reshape(SKV, D).astype(jnp.bfloat16)

    out = pl.pallas_call(
        _body,
        out_shape=jax.ShapeDtypeStruct((SQ, D), jnp.float32),
        in_specs=[pl.BlockSpec(memory_space=pltpu.VMEM)] * 5,
        out_specs=pl.BlockSpec(memory_space=pltpu.VMEM),
        scratch_shapes=[
            pltpu.VMEM((N_DEV * SQ, D), jnp.bfloat16),
            pltpu.SemaphoreType.DMA((N_DEV - 1,)),
            pltpu.SemaphoreType.DMA((N_DEV - 1,)),
            pltpu.VMEM((3, SQ, D), jnp.bfloat16),
            pltpu.VMEM((3, SQ, 128), jnp.float32),
            pltpu.VMEM((3, SQ, D), jnp.bfloat16),
            pltpu.VMEM((3, SQ, 128), jnp.float32),
            pltpu.SemaphoreType.DMA((3,)),
            pltpu.SemaphoreType.DMA((3,)),
            pltpu.SemaphoreType.DMA((3,)),
            pltpu.SemaphoreType.DMA((3,)),
            pltpu.VMEM((SQ, D), jnp.float32),
            pltpu.VMEM((SQ, 128), jnp.float32),
        ],
        compiler_params=pltpu.CompilerParams(
            collective_id=0,
            vmem_limit_bytes=100 * 1024 * 1024,
        ),
    )(xb, wqb, wob, kb, vb)
    return out.reshape(1, SQ, D)
